# baseline (speedup 1.0000x reference)
import functools

import numpy as np
import jax
import jax.numpy as jnp
from jax import lax
from jax.experimental import pallas as pl
from jax.experimental.pallas import tpu as pltpu
from jax.experimental.pallas import tpu_sc as plsc

N_EMBED = 128
TIMESTEPS = 1000
BATCH = 16384


def _build_table() -> np.ndarray:
    half = N_EMBED // 2
    b = (np.arange(TIMESTEPS, dtype=np.float32) / np.float32(10000.0))[:, None]
    e = (np.arange(half, dtype=np.float32) / np.float32(N_EMBED))[None, :]
    base = np.power(b, e, dtype=np.float32)
    emb = np.stack((np.sin(base), np.cos(base)), axis=-1).reshape(
        TIMESTEPS, N_EMBED
    )
    return emb.astype(np.float32)


_TABLE = _build_table()

_INFO = plsc.get_sparse_core_info()
_NC = _INFO.num_cores
_NS = _INFO.num_subcores
_NW = _NC * _NS
_B_PER_W = BATCH // _NW
_CHUNK = 256
_NCHUNK = _B_PER_W // _CHUNK


_mesh = plsc.VectorSubcoreMesh(core_axis_name="c", subcore_axis_name="s")


@functools.partial(
    pl.kernel,
    mesh=_mesh,
    out_type=jax.ShapeDtypeStruct((BATCH, N_EMBED), jnp.float32),
    scratch_types=[
        pltpu.VMEM_SHARED((TIMESTEPS, N_EMBED), jnp.float32),
        pltpu.VMEM((_B_PER_W,), jnp.int32),
        pltpu.VMEM((_B_PER_W, N_EMBED), jnp.float32),
    ]
    + [pltpu.SemaphoreType.DMA] * _NCHUNK
    + [pltpu.SemaphoreType.DMA],
)
def _gather_kernel(table_hbm, t_hbm, out_hbm, table_sp, idx_v, rows_v, *sems):
    gsems, osem = sems[:_NCHUNK], sems[_NCHUNK]
    sid = lax.axis_index("s")
    wid = sid * _NC + lax.axis_index("c")
    base = wid * _B_PER_W

    @pl.when(sid == 0)
    def _stage_table():
        pltpu.sync_copy(table_hbm, table_sp)

    pltpu.sync_copy(t_hbm.at[pl.ds(base, _B_PER_W)], idx_v)
    plsc.subcore_barrier()

    gathers = []
    for j in range(_NCHUNK):
        gathers.append(
            pltpu.async_copy(
                table_sp.at[idx_v.at[pl.ds(j * _CHUNK, _CHUNK)]],
                rows_v.at[pl.ds(j * _CHUNK, _CHUNK)],
                gsems[j],
            )
        )
    outs = []
    for j in range(_NCHUNK):
        gathers[j].wait()
        outs.append(
            pltpu.async_copy(
                rows_v.at[pl.ds(j * _CHUNK, _CHUNK)],
                out_hbm.at[pl.ds(base + j * _CHUNK, _CHUNK)],
                osem,
            )
        )
    for o in outs:
        o.wait()


def kernel(t):
    table = jnp.asarray(_TABLE)
    return _gather_kernel(table, t)


# speedup vs baseline: 1.0350x; 1.0350x over previous
import functools

import numpy as np
import jax
import jax.numpy as jnp
from jax import lax
from jax.experimental import pallas as pl
from jax.experimental.pallas import tpu as pltpu
from jax.experimental.pallas import tpu_sc as plsc

N_EMBED = 128
TIMESTEPS = 1000
BATCH = 16384


def _build_table() -> np.ndarray:
    half = N_EMBED // 2
    b = (np.arange(TIMESTEPS, dtype=np.float32) / np.float32(10000.0))[:, None]
    e = (np.arange(half, dtype=np.float32) / np.float32(N_EMBED))[None, :]
    base = np.power(b, e, dtype=np.float32)
    emb = np.stack((np.sin(base), np.cos(base)), axis=-1).reshape(
        TIMESTEPS, N_EMBED
    )
    return emb.astype(np.float32)


_TABLE = _build_table()

_INFO = plsc.get_sparse_core_info()
_NC = _INFO.num_cores
_NS = _INFO.num_subcores
_NW = _NC * _NS
_B_PER_W = BATCH // _NW
_CHUNK = 64
_NCHUNK = _B_PER_W // _CHUNK


_mesh = plsc.VectorSubcoreMesh(core_axis_name="c", subcore_axis_name="s")


@functools.partial(
    pl.kernel,
    mesh=_mesh,
    out_type=jax.ShapeDtypeStruct((BATCH, N_EMBED), jnp.float32),
    scratch_types=[
        pltpu.VMEM_SHARED((TIMESTEPS, N_EMBED), jnp.float32),
        pltpu.VMEM((_B_PER_W,), jnp.int32),
        pltpu.VMEM((_B_PER_W, N_EMBED), jnp.float32),
    ]
    + [pltpu.SemaphoreType.DMA] * _NCHUNK
    + [pltpu.SemaphoreType.DMA],
)
def _gather_kernel(table_hbm, t_hbm, out_hbm, table_sp, idx_v, rows_v, *sems):
    gsems, osem = sems[:_NCHUNK], sems[_NCHUNK]
    sid = lax.axis_index("s")
    wid = sid * _NC + lax.axis_index("c")
    base = wid * _B_PER_W

    @pl.when(sid == 0)
    def _stage_table():
        pltpu.sync_copy(table_hbm, table_sp)

    pltpu.sync_copy(t_hbm.at[pl.ds(base, _B_PER_W)], idx_v)
    plsc.subcore_barrier()

    gathers = []
    for j in range(_NCHUNK):
        gathers.append(
            pltpu.async_copy(
                table_sp.at[idx_v.at[pl.ds(j * _CHUNK, _CHUNK)]],
                rows_v.at[pl.ds(j * _CHUNK, _CHUNK)],
                gsems[j],
            )
        )
    outs = []
    for j in range(_NCHUNK):
        gathers[j].wait()
        outs.append(
            pltpu.async_copy(
                rows_v.at[pl.ds(j * _CHUNK, _CHUNK)],
                out_hbm.at[pl.ds(base + j * _CHUNK, _CHUNK)],
                osem,
            )
        )
    for o in outs:
        o.wait()


def kernel(t):
    table = jnp.asarray(_TABLE)
    return _gather_kernel(table, t)
